# baseline (speedup 1.0000x reference)
import functools

import jax
import jax.numpy as jnp
from jax import lax
from jax.experimental import pallas as pl
from jax.experimental.pallas import tpu as pltpu
from jax.experimental.pallas import tpu_sc as plsc

CHUNK = 128
NSLOT_ID = 4
NSLOT_EXP = 3


@functools.lru_cache(maxsize=None)
def _make_sc_gather(V: int, D: int, B: int):
    info = plsc.get_sparse_core_info()
    NC, NS = info.num_cores, info.num_subcores
    NW = NC * NS
    b_per_w = B // NW
    n_chunks = b_per_w // CHUNK
    mesh = plsc.VectorSubcoreMesh(core_axis_name="c", subcore_axis_name="s")

    @functools.partial(
        pl.kernel,
        mesh=mesh,
        out_type=(
            jax.ShapeDtypeStruct((B, D), jnp.float32),
            jax.ShapeDtypeStruct((B, D), jnp.float32),
        ),
        scratch_types=[
            pltpu.VMEM((n_chunks, CHUNK), jnp.int32),
            pltpu.VMEM((NSLOT_ID, CHUNK, D), jnp.float32),
            pltpu.VMEM((NSLOT_EXP, CHUNK, D), jnp.float32),
        ]
        + [pltpu.SemaphoreType.DMA] * (2 * (NSLOT_ID + NSLOT_EXP)),
    )
    def gather_kernel(id_hbm, exp_hbm, idx_hbm, id_out, exp_out,
                      idx_v, buf_id, buf_exp, *sems):
        wid = lax.axis_index("s") * NC + lax.axis_index("c")
        base = wid * b_per_w
        sg_id = sems[:NSLOT_ID]
        sg_exp = sems[NSLOT_ID:NSLOT_ID + NSLOT_EXP]
        sw_id = sems[NSLOT_ID + NSLOT_EXP:2 * NSLOT_ID + NSLOT_EXP]
        sw_exp = sems[2 * NSLOT_ID + NSLOT_EXP:]

        def issue_gather(j):
            si, se = j % NSLOT_ID, j % NSLOT_EXP
            return (
                pltpu.async_copy(id_hbm.at[idx_v.at[j]], buf_id.at[si], sg_id[si]),
                pltpu.async_copy(exp_hbm.at[idx_v.at[j]], buf_exp.at[se], sg_exp[se]),
            )

        pltpu.sync_copy(idx_hbm.at[pl.ds(wid * n_chunks, n_chunks)], idx_v)
        inflight = [issue_gather(j)
                    for j in range(min(NSLOT_EXP, n_chunks))]

        writes = [None] * n_chunks
        for j in range(n_chunks):
            si, se = j % NSLOT_ID, j % NSLOT_EXP
            row0 = base + j * CHUNK
            cp_id, cp_exp = inflight[j]
            cp_id.wait()
            w_id = pltpu.async_copy(buf_id.at[si], id_out.at[pl.ds(row0, CHUNK)], sw_id[si])
            cp_exp.wait()
            w_exp = pltpu.async_copy(buf_exp.at[se], exp_out.at[pl.ds(row0, CHUNK)], sw_exp[se])
            writes[j] = (w_id, w_exp)
            k = j + NSLOT_EXP
            if k < n_chunks:
                w_exp.wait()
                writes[j] = (w_id,)
                inflight.append(issue_gather(k))
        for j in range(n_chunks):
            for c in writes[j]:
                c.wait()

    return gather_kernel


def kernel(id_table, exp_table, indices):
    (B,) = indices.shape
    V, D = id_table.shape
    idx2d = indices.astype(jnp.int32).reshape(B // CHUNK, CHUNK)
    f = _make_sc_gather(V, D, B)
    return f(id_table, exp_table, idx2d)


# speedup vs baseline: 1.0125x; 1.0125x over previous
import functools

import jax
import jax.numpy as jnp
from jax import lax
from jax.experimental import pallas as pl
from jax.experimental.pallas import tpu as pltpu
from jax.experimental.pallas import tpu_sc as plsc

CHUNK = 128
NSLOT_ID = 4
NSLOT_EXP = 3


@functools.lru_cache(maxsize=None)
def _make_sc_gather(V: int, D: int, B: int):
    info = plsc.get_sparse_core_info()
    NC, NS = info.num_cores, info.num_subcores
    NW = NC * NS
    b_per_w = B // NW
    n_chunks = b_per_w // CHUNK
    mesh = plsc.VectorSubcoreMesh(core_axis_name="c", subcore_axis_name="s")

    @functools.partial(
        pl.kernel,
        mesh=mesh,
        out_type=(
            jax.ShapeDtypeStruct((B, D), jnp.float32),
            jax.ShapeDtypeStruct((B, D), jnp.float32),
        ),
        scratch_types=[
            pltpu.VMEM((b_per_w,), jnp.int32),
            pltpu.VMEM((NSLOT_ID, CHUNK, D), jnp.float32),
            pltpu.VMEM((NSLOT_EXP, CHUNK, D), jnp.float32),
        ]
        + [pltpu.SemaphoreType.DMA] * (2 * (NSLOT_ID + NSLOT_EXP)),
    )
    def gather_kernel(id_hbm, exp_hbm, idx_hbm, id_out, exp_out,
                      idx_v, buf_id, buf_exp, *sems):
        wid = lax.axis_index("s") * NC + lax.axis_index("c")
        base = wid * b_per_w
        sg_id = sems[:NSLOT_ID]
        sg_exp = sems[NSLOT_ID:NSLOT_ID + NSLOT_EXP]
        sw_id = sems[NSLOT_ID + NSLOT_EXP:2 * NSLOT_ID + NSLOT_EXP]
        sw_exp = sems[2 * NSLOT_ID + NSLOT_EXP:]

        def issue_gather(j):
            si, se = j % NSLOT_ID, j % NSLOT_EXP
            idx_j = idx_v.at[pl.ds(j * CHUNK, CHUNK)]
            return (
                pltpu.async_copy(id_hbm.at[idx_j], buf_id.at[si], sg_id[si]),
                pltpu.async_copy(exp_hbm.at[idx_j], buf_exp.at[se], sg_exp[se]),
            )

        pltpu.sync_copy(idx_hbm.at[pl.ds(base, b_per_w)], idx_v)
        inflight = [issue_gather(j)
                    for j in range(min(NSLOT_EXP, n_chunks))]

        writes = [None] * n_chunks
        for j in range(n_chunks):
            si, se = j % NSLOT_ID, j % NSLOT_EXP
            row0 = base + j * CHUNK
            cp_id, cp_exp = inflight[j]
            cp_id.wait()
            w_id = pltpu.async_copy(buf_id.at[si], id_out.at[pl.ds(row0, CHUNK)], sw_id[si])
            cp_exp.wait()
            w_exp = pltpu.async_copy(buf_exp.at[se], exp_out.at[pl.ds(row0, CHUNK)], sw_exp[se])
            writes[j] = (w_id, w_exp)
            k = j + NSLOT_EXP
            if k < n_chunks:
                w_exp.wait()
                writes[j] = (w_id,)
                inflight.append(issue_gather(k))
        for j in range(n_chunks):
            for c in writes[j]:
                c.wait()

    return gather_kernel


def kernel(id_table, exp_table, indices):
    (B,) = indices.shape
    V, D = id_table.shape
    f = _make_sc_gather(V, D, B)
    return f(id_table, exp_table, indices.astype(jnp.int32))
